# baseline (speedup 1.0000x reference)
import jax
import jax.numpy as jnp
from jax import lax
from jax.experimental import pallas as pl
from jax.experimental.pallas import tpu as pltpu
from jax.experimental.pallas import tpu_sc as plsc

EMB_DIM = 128
BATCH = 16384
NC, NS = 2, 16
NW = NC * NS
B_PER_W = BATCH // NW
CHUNK = 64
N_CHUNKS = B_PER_W // CHUNK
GROUPS = CHUNK // 16
NBUF = 6


def _sc_body(ne, ce, si, ti, lab, out_hbm, idx_s, idx_t, lab_v, srows, trows,
             out_v, sem0, sem1, sem2, sem3, sem4, sem5):
    wid = lax.axis_index("s") * NC + lax.axis_index("c")
    base = wid * B_PER_W
    pltpu.sync_copy(si.at[pl.ds(base, B_PER_W)], idx_s)
    pltpu.sync_copy(ti.at[pl.ds(base, B_PER_W)], idx_t)
    pltpu.sync_copy(lab.at[pl.ds(base, B_PER_W)], lab_v)
    sems = (sem0, sem1, sem2, sem3, sem4, sem5)
    copies = [None] * NBUF

    def start(chunk):
        buf = chunk % NBUF
        sl = pl.ds(chunk * CHUNK, CHUNK)
        copies[buf] = (
            pltpu.async_copy(ne.at[idx_s.at[sl]], srows.at[buf], sems[buf]),
            pltpu.async_copy(ce.at[idx_t.at[sl]], trows.at[buf], sems[buf]),
        )

    lane = lax.iota(jnp.int32, 16)
    rowbases = [g * 16 + lane for g in range(GROUPS)]
    zeros = tuple(jnp.zeros((16,), jnp.float32) for _ in range(GROUPS))

    for chunk in range(min(NBUF, N_CHUNKS)):
        start(chunk)
    for chunk in range(N_CHUNKS):
        buf = chunk % NBUF
        cp_s, cp_t = copies[buf]
        cp_s.wait()
        cp_t.wait()
        sbuf = srows.at[buf]
        tbuf = trows.at[buf]

        @plsc.parallel_loop(0, EMB_DIM, unroll=2, carry=zeros)
        def accs(d, accs_in, sbuf=sbuf, tbuf=tbuf):
            col = (jnp.broadcast_to(d, (16,)).astype(jnp.int32) + lane) & 127
            out = []
            for g in range(GROUPS):
                s = plsc.load_gather(sbuf, [rowbases[g], col])
                t = plsc.load_gather(tbuf, [rowbases[g], col])
                out.append(accs_in[g] + s * t)
            return tuple(out)

        if chunk + NBUF < N_CHUNKS:
            start(chunk + NBUF)
        for g in range(GROUPS):
            sl = pl.ds(chunk * CHUNK + g * 16, 16)
            out_v[sl] = accs[g] * lab_v[sl]
    pltpu.sync_copy(out_v, out_hbm.at[wid])


_sc_inner = pl.kernel(
    _sc_body,
    out_type=jax.ShapeDtypeStruct((NW, B_PER_W), jnp.float32),
    mesh=plsc.VectorSubcoreMesh(core_axis_name="c", subcore_axis_name="s"),
    scratch_types=[
        pltpu.VMEM((B_PER_W,), jnp.int32),
        pltpu.VMEM((B_PER_W,), jnp.int32),
        pltpu.VMEM((B_PER_W,), jnp.float32),
        pltpu.VMEM((NBUF, CHUNK, EMB_DIM), jnp.float32),
        pltpu.VMEM((NBUF, CHUNK, EMB_DIM), jnp.float32),
        pltpu.VMEM((B_PER_W,), jnp.float32),
        pltpu.SemaphoreType.DMA,
        pltpu.SemaphoreType.DMA,
        pltpu.SemaphoreType.DMA,
        pltpu.SemaphoreType.DMA,
        pltpu.SemaphoreType.DMA,
        pltpu.SemaphoreType.DMA,
    ],
    compiler_params=pltpu.CompilerParams(needs_layout_passes=False),
)


def _loss_body(x_ref, o_ref):
    x = x_ref[...]
    ls = jnp.minimum(x, 0.0) - jnp.log1p(jnp.exp(-jnp.abs(x)))
    o_ref[0, 0] = -jnp.sum(ls) * (1.0 / BATCH)


_loss = pl.pallas_call(
    _loss_body,
    out_shape=jax.ShapeDtypeStruct((1, 1), jnp.float32),
    out_specs=pl.BlockSpec(memory_space=pltpu.SMEM),
)


def kernel(source_node, target_node, label, nodes_embed, context_nodes_embed):
    si = source_node.astype(jnp.int32)
    ti = target_node.astype(jnp.int32)
    pos = _sc_inner(nodes_embed, context_nodes_embed, si, ti, label)
    loss = _loss(pos)
    return loss[0, 0]


# speedup vs baseline: 1.0161x; 1.0161x over previous
import jax
import jax.numpy as jnp
from jax import lax
from jax.experimental import pallas as pl
from jax.experimental.pallas import tpu as pltpu
from jax.experimental.pallas import tpu_sc as plsc

EMB_DIM = 128
BATCH = 16384
NC, NS = 2, 16
NW = NC * NS
B_PER_W = BATCH // NW
CHUNK = 32
N_CHUNKS = B_PER_W // CHUNK
GROUPS = CHUNK // 16
NBUF = 8


def _sc_body(ne, ce, si, ti, lab, out_hbm, idx_s, idx_t, lab_v, srows, trows,
             out_v, sem0, sem1, sem2, sem3, sem4, sem5, sem6, sem7):
    wid = lax.axis_index("s") * NC + lax.axis_index("c")
    base = wid * B_PER_W
    pltpu.sync_copy(si.at[pl.ds(base, B_PER_W)], idx_s)
    pltpu.sync_copy(ti.at[pl.ds(base, B_PER_W)], idx_t)
    pltpu.sync_copy(lab.at[pl.ds(base, B_PER_W)], lab_v)
    sems = (sem0, sem1, sem2, sem3, sem4, sem5, sem6, sem7)
    copies = [None] * NBUF

    def start(chunk):
        buf = chunk % NBUF
        sl = pl.ds(chunk * CHUNK, CHUNK)
        copies[buf] = (
            pltpu.async_copy(ne.at[idx_s.at[sl]], srows.at[buf], sems[buf]),
            pltpu.async_copy(ce.at[idx_t.at[sl]], trows.at[buf], sems[buf]),
        )

    lane = lax.iota(jnp.int32, 16)
    rowbases = [g * 16 + lane for g in range(GROUPS)]
    zeros = tuple(jnp.zeros((16,), jnp.float32) for _ in range(GROUPS))

    for chunk in range(min(NBUF, N_CHUNKS)):
        start(chunk)
    for chunk in range(N_CHUNKS):
        buf = chunk % NBUF
        cp_s, cp_t = copies[buf]
        cp_s.wait()
        cp_t.wait()
        sbuf = srows.at[buf]
        tbuf = trows.at[buf]

        @plsc.parallel_loop(0, EMB_DIM, unroll=2, carry=zeros)
        def accs(d, accs_in, sbuf=sbuf, tbuf=tbuf):
            col = (jnp.broadcast_to(d, (16,)).astype(jnp.int32) + lane) & 127
            out = []
            for g in range(GROUPS):
                s = plsc.load_gather(sbuf, [rowbases[g], col])
                t = plsc.load_gather(tbuf, [rowbases[g], col])
                out.append(accs_in[g] + s * t)
            return tuple(out)

        if chunk + NBUF < N_CHUNKS:
            start(chunk + NBUF)
        for g in range(GROUPS):
            sl = pl.ds(chunk * CHUNK + g * 16, 16)
            out_v[sl] = accs[g] * lab_v[sl]
    pltpu.sync_copy(out_v, out_hbm.at[wid])


_sc_inner = pl.kernel(
    _sc_body,
    out_type=jax.ShapeDtypeStruct((NW, B_PER_W), jnp.float32),
    mesh=plsc.VectorSubcoreMesh(core_axis_name="c", subcore_axis_name="s"),
    scratch_types=[
        pltpu.VMEM((B_PER_W,), jnp.int32),
        pltpu.VMEM((B_PER_W,), jnp.int32),
        pltpu.VMEM((B_PER_W,), jnp.float32),
        pltpu.VMEM((NBUF, CHUNK, EMB_DIM), jnp.float32),
        pltpu.VMEM((NBUF, CHUNK, EMB_DIM), jnp.float32),
        pltpu.VMEM((B_PER_W,), jnp.float32),
        pltpu.SemaphoreType.DMA,
        pltpu.SemaphoreType.DMA,
        pltpu.SemaphoreType.DMA,
        pltpu.SemaphoreType.DMA,
        pltpu.SemaphoreType.DMA,
        pltpu.SemaphoreType.DMA,
        pltpu.SemaphoreType.DMA,
        pltpu.SemaphoreType.DMA,
    ],
    compiler_params=pltpu.CompilerParams(needs_layout_passes=False),
)


def _loss_body(x_ref, o_ref):
    x = x_ref[...]
    ls = jnp.minimum(x, 0.0) - jnp.log1p(jnp.exp(-jnp.abs(x)))
    o_ref[0, 0] = -jnp.sum(ls) * (1.0 / BATCH)


_loss = pl.pallas_call(
    _loss_body,
    out_shape=jax.ShapeDtypeStruct((1, 1), jnp.float32),
    out_specs=pl.BlockSpec(memory_space=pltpu.SMEM),
)


def kernel(source_node, target_node, label, nodes_embed, context_nodes_embed):
    si = source_node.astype(jnp.int32)
    ti = target_node.astype(jnp.int32)
    pos = _sc_inner(nodes_embed, context_nodes_embed, si, ti, label)
    loss = _loss(pos)
    return loss[0, 0]


# speedup vs baseline: 1.0188x; 1.0026x over previous
import jax
import jax.numpy as jnp
from jax import lax
from jax.experimental import pallas as pl
from jax.experimental.pallas import tpu as pltpu
from jax.experimental.pallas import tpu_sc as plsc

EMB_DIM = 128
BATCH = 16384
NC, NS = 2, 16
NW = NC * NS
B_PER_W = BATCH // NW
CHUNK = 32
N_CHUNKS = B_PER_W // CHUNK
GROUPS = CHUNK // 16
NBUF = 8


def _sc_body(ne, ce, si, ti, lab, out_hbm, idx_s, idx_t, lab_v, srows, trows,
             out_v, sem0, sem1, sem2, sem3, sem4, sem5, sem6, sem7):
    wid = lax.axis_index("s") * NC + lax.axis_index("c")
    base = wid * B_PER_W
    pltpu.sync_copy(si.at[pl.ds(base, B_PER_W)], idx_s)
    pltpu.sync_copy(ti.at[pl.ds(base, B_PER_W)], idx_t)
    pltpu.sync_copy(lab.at[pl.ds(base, B_PER_W)], lab_v)
    sems = (sem0, sem1, sem2, sem3, sem4, sem5, sem6, sem7)
    copies = [None] * NBUF

    def start(chunk):
        buf = chunk % NBUF
        sl = pl.ds(chunk * CHUNK, CHUNK)
        copies[buf] = (
            pltpu.async_copy(ne.at[idx_s.at[sl]], srows.at[buf], sems[buf]),
            pltpu.async_copy(ce.at[idx_t.at[sl]], trows.at[buf], sems[buf]),
        )

    lane = lax.iota(jnp.int32, 16)
    rowbases = [g * 16 + lane for g in range(GROUPS)]
    zeros = tuple(jnp.zeros((16,), jnp.float32) for _ in range(GROUPS))

    for chunk in range(min(NBUF, N_CHUNKS)):
        start(chunk)
    for chunk in range(N_CHUNKS):
        buf = chunk % NBUF
        cp_s, cp_t = copies[buf]
        cp_s.wait()
        cp_t.wait()
        sbuf = srows.at[buf]
        tbuf = trows.at[buf]

        @plsc.parallel_loop(0, EMB_DIM, unroll=2, carry=zeros)
        def accs(d, accs_in, sbuf=sbuf, tbuf=tbuf):
            col = (jnp.broadcast_to(d, (16,)).astype(jnp.int32) + lane) & 127
            out = []
            for g in range(GROUPS):
                s = plsc.load_gather(sbuf, [rowbases[g], col])
                t = plsc.load_gather(tbuf, [rowbases[g], col])
                out.append(accs_in[g] + s * t)
            return tuple(out)

        if chunk + NBUF < N_CHUNKS:
            start(chunk + NBUF)
        for g in range(GROUPS):
            sl = pl.ds(chunk * CHUNK + g * 16, 16)
            out_v[sl] = accs[g] * lab_v[sl]
    pltpu.sync_copy(out_v, out_hbm.at[wid])


_sc_inner = pl.kernel(
    _sc_body,
    out_type=jax.ShapeDtypeStruct((NW, B_PER_W), jnp.float32),
    mesh=plsc.VectorSubcoreMesh(core_axis_name="c", subcore_axis_name="s"),
    scratch_types=[
        pltpu.VMEM((B_PER_W,), jnp.int32),
        pltpu.VMEM((B_PER_W,), jnp.int32),
        pltpu.VMEM((B_PER_W,), jnp.float32),
        pltpu.VMEM((NBUF, CHUNK, EMB_DIM), jnp.float32),
        pltpu.VMEM((NBUF, CHUNK, EMB_DIM), jnp.float32),
        pltpu.VMEM((B_PER_W,), jnp.float32),
        pltpu.SemaphoreType.DMA,
        pltpu.SemaphoreType.DMA,
        pltpu.SemaphoreType.DMA,
        pltpu.SemaphoreType.DMA,
        pltpu.SemaphoreType.DMA,
        pltpu.SemaphoreType.DMA,
        pltpu.SemaphoreType.DMA,
        pltpu.SemaphoreType.DMA,
    ],
    compiler_params=pltpu.CompilerParams(needs_layout_passes=False, skip_device_barrier=True),
)


def _loss_body(x_ref, o_ref):
    x = x_ref[...]
    ls = jnp.minimum(x, 0.0) - jnp.log1p(jnp.exp(-jnp.abs(x)))
    o_ref[0, 0] = -jnp.sum(ls) * (1.0 / BATCH)


_loss = pl.pallas_call(
    _loss_body,
    out_shape=jax.ShapeDtypeStruct((1, 1), jnp.float32),
    out_specs=pl.BlockSpec(memory_space=pltpu.SMEM),
)


def kernel(source_node, target_node, label, nodes_embed, context_nodes_embed):
    si = source_node.astype(jnp.int32)
    ti = target_node.astype(jnp.int32)
    pos = _sc_inner(nodes_embed, context_nodes_embed, si, ti, label)
    loss = _loss(pos)
    return loss[0, 0]


# speedup vs baseline: 1.0204x; 1.0016x over previous
import jax
import jax.numpy as jnp
from jax import lax
from jax.experimental import pallas as pl
from jax.experimental.pallas import tpu as pltpu
from jax.experimental.pallas import tpu_sc as plsc

EMB_DIM = 128
BATCH = 16384
NC, NS = 2, 16
NW = NC * NS
B_PER_W = BATCH // NW
CHUNK = 32
N_CHUNKS = B_PER_W // CHUNK
GROUPS = CHUNK // 16
NBUF = 8


def _sc_body(ne, ce, si, ti, lab, out_hbm, idx_s, idx_t, lab_v, srows, trows,
             out_v, sem0, sem1, sem2, sem3, sem4, sem5, sem6, sem7):
    wid = lax.axis_index("s") * NC + lax.axis_index("c")
    base = wid * B_PER_W
    pltpu.sync_copy(si.at[pl.ds(base, B_PER_W)], idx_s)
    pltpu.sync_copy(ti.at[pl.ds(base, B_PER_W)], idx_t)
    pltpu.sync_copy(lab.at[pl.ds(base, B_PER_W)], lab_v)
    sems = (sem0, sem1, sem2, sem3, sem4, sem5, sem6, sem7)
    copies = [None] * NBUF

    def start(chunk):
        buf = chunk % NBUF
        sl = pl.ds(chunk * CHUNK, CHUNK)
        copies[buf] = (
            pltpu.async_copy(ne.at[idx_s.at[sl]], srows.at[buf], sems[buf]),
            pltpu.async_copy(ce.at[idx_t.at[sl]], trows.at[buf], sems[buf]),
        )

    lane = lax.iota(jnp.int32, 16)
    rowbases = [g * 16 + lane for g in range(GROUPS)]
    zeros = tuple(jnp.zeros((16,), jnp.float32) for _ in range(GROUPS))

    for chunk in range(min(NBUF, N_CHUNKS)):
        start(chunk)
    for chunk in range(N_CHUNKS):
        buf = chunk % NBUF
        cp_s, cp_t = copies[buf]
        cp_s.wait()
        cp_t.wait()
        sbuf = srows.at[buf]
        tbuf = trows.at[buf]

        @plsc.parallel_loop(0, EMB_DIM, unroll=2, carry=zeros)
        def accs(d, accs_in, sbuf=sbuf, tbuf=tbuf):
            col = (jnp.broadcast_to(d, (16,)).astype(jnp.int32) + lane) & 127
            out = []
            for g in range(GROUPS):
                s = plsc.load_gather(sbuf, [rowbases[g], col])
                t = plsc.load_gather(tbuf, [rowbases[g], col])
                out.append(accs_in[g] + s * t)
            return tuple(out)

        if chunk + NBUF < N_CHUNKS:
            start(chunk + NBUF)
        for g in range(GROUPS):
            sl = pl.ds(chunk * CHUNK + g * 16, 16)
            out_v[sl] = accs[g] * lab_v[sl]
    pltpu.sync_copy(out_v, out_hbm.at[wid])


_sc_inner = pl.kernel(
    _sc_body,
    out_type=jax.ShapeDtypeStruct((NW, B_PER_W), jnp.float32),
    mesh=plsc.VectorSubcoreMesh(core_axis_name="c", subcore_axis_name="s"),
    scratch_types=[
        pltpu.VMEM((B_PER_W,), jnp.int32),
        pltpu.VMEM((B_PER_W,), jnp.int32),
        pltpu.VMEM((B_PER_W,), jnp.float32),
        pltpu.VMEM((NBUF, CHUNK, EMB_DIM), jnp.float32),
        pltpu.VMEM((NBUF, CHUNK, EMB_DIM), jnp.float32),
        pltpu.VMEM((B_PER_W,), jnp.float32),
        pltpu.SemaphoreType.DMA,
        pltpu.SemaphoreType.DMA,
        pltpu.SemaphoreType.DMA,
        pltpu.SemaphoreType.DMA,
        pltpu.SemaphoreType.DMA,
        pltpu.SemaphoreType.DMA,
        pltpu.SemaphoreType.DMA,
        pltpu.SemaphoreType.DMA,
    ],
    compiler_params=pltpu.CompilerParams(needs_layout_passes=False, disable_bounds_checks=True),
)


def _loss_body(x_ref, o_ref):
    x = x_ref[...]
    ls = jnp.minimum(x, 0.0) - jnp.log1p(jnp.exp(-jnp.abs(x)))
    o_ref[0, 0] = -jnp.sum(ls) * (1.0 / BATCH)


_loss = pl.pallas_call(
    _loss_body,
    out_shape=jax.ShapeDtypeStruct((1, 1), jnp.float32),
    out_specs=pl.BlockSpec(memory_space=pltpu.SMEM),
)


def kernel(source_node, target_node, label, nodes_embed, context_nodes_embed):
    si = source_node.astype(jnp.int32)
    ti = target_node.astype(jnp.int32)
    pos = _sc_inner(nodes_embed, context_nodes_embed, si, ti, label)
    loss = _loss(pos)
    return loss[0, 0]


# speedup vs baseline: 1.0547x; 1.0336x over previous
import jax
import jax.numpy as jnp
from jax import lax
from jax.experimental import pallas as pl
from jax.experimental.pallas import tpu as pltpu
from jax.experimental.pallas import tpu_sc as plsc

EMB_DIM = 128
BATCH = 16384
NC, NS = 2, 16
NW = NC * NS
B_PER_W = BATCH // NW
CHUNK = 32
N_CHUNKS = B_PER_W // CHUNK
GROUPS = CHUNK // 16
NBUF = 8


def _sc_body(ne, ce, si, ti, lab, out_hbm, idx_s, idx_t, lab_v, srows, trows,
             out_v, sem_stage, sem0, sem1, sem2, sem3, sem4, sem5, sem6, sem7):
    wid = lax.axis_index("s") * NC + lax.axis_index("c")
    base = wid * B_PER_W
    cp_i = pltpu.async_copy(si.at[pl.ds(base, B_PER_W)], idx_s, sem_stage)
    cp_j = pltpu.async_copy(ti.at[pl.ds(base, B_PER_W)], idx_t, sem_stage)
    cp_l = pltpu.async_copy(lab.at[pl.ds(base, B_PER_W)], lab_v, sem_stage)
    cp_i.wait()
    cp_j.wait()
    cp_l.wait()
    sems = (sem0, sem1, sem2, sem3, sem4, sem5, sem6, sem7)
    copies = [None] * NBUF

    def start(chunk):
        buf = chunk % NBUF
        sl = pl.ds(chunk * CHUNK, CHUNK)
        copies[buf] = (
            pltpu.async_copy(ne.at[idx_s.at[sl]], srows.at[buf], sems[buf]),
            pltpu.async_copy(ce.at[idx_t.at[sl]], trows.at[buf], sems[buf]),
        )

    lane = lax.iota(jnp.int32, 16)
    rowbases = [g * 16 + lane for g in range(GROUPS)]
    zeros = tuple(jnp.zeros((16,), jnp.float32) for _ in range(GROUPS))

    for chunk in range(min(NBUF, N_CHUNKS)):
        start(chunk)
    for chunk in range(N_CHUNKS):
        buf = chunk % NBUF
        cp_s, cp_t = copies[buf]
        cp_s.wait()
        cp_t.wait()
        sbuf = srows.at[buf]
        tbuf = trows.at[buf]

        @plsc.parallel_loop(0, EMB_DIM, unroll=2, carry=zeros)
        def accs(d, accs_in, sbuf=sbuf, tbuf=tbuf):
            col = (jnp.broadcast_to(d, (16,)).astype(jnp.int32) + lane) & 127
            out = []
            for g in range(GROUPS):
                s = plsc.load_gather(sbuf, [rowbases[g], col])
                t = plsc.load_gather(tbuf, [rowbases[g], col])
                out.append(accs_in[g] + s * t)
            return tuple(out)

        if chunk + NBUF < N_CHUNKS:
            start(chunk + NBUF)
        for g in range(GROUPS):
            sl = pl.ds(chunk * CHUNK + g * 16, 16)
            out_v[sl] = accs[g] * lab_v[sl]
    pltpu.sync_copy(out_v, out_hbm.at[wid])


_sc_inner = pl.kernel(
    _sc_body,
    out_type=jax.ShapeDtypeStruct((NW, B_PER_W), jnp.float32),
    mesh=plsc.VectorSubcoreMesh(core_axis_name="c", subcore_axis_name="s"),
    scratch_types=[
        pltpu.VMEM((B_PER_W,), jnp.int32),
        pltpu.VMEM((B_PER_W,), jnp.int32),
        pltpu.VMEM((B_PER_W,), jnp.float32),
        pltpu.VMEM((NBUF, CHUNK, EMB_DIM), jnp.float32),
        pltpu.VMEM((NBUF, CHUNK, EMB_DIM), jnp.float32),
        pltpu.VMEM((B_PER_W,), jnp.float32),
        pltpu.SemaphoreType.DMA,
        pltpu.SemaphoreType.DMA,
        pltpu.SemaphoreType.DMA,
        pltpu.SemaphoreType.DMA,
        pltpu.SemaphoreType.DMA,
        pltpu.SemaphoreType.DMA,
        pltpu.SemaphoreType.DMA,
        pltpu.SemaphoreType.DMA,
        pltpu.SemaphoreType.DMA,
    ],
    compiler_params=pltpu.CompilerParams(needs_layout_passes=False, disable_bounds_checks=True),
)


def _loss_body(x_ref, o_ref):
    x = x_ref[...]
    ls = jnp.minimum(x, 0.0) - jnp.log1p(jnp.exp(-jnp.abs(x)))
    o_ref[0, 0] = -jnp.sum(ls) * (1.0 / BATCH)


_loss = pl.pallas_call(
    _loss_body,
    out_shape=jax.ShapeDtypeStruct((1, 1), jnp.float32),
    out_specs=pl.BlockSpec(memory_space=pltpu.SMEM),
)


def kernel(source_node, target_node, label, nodes_embed, context_nodes_embed):
    si = source_node.astype(jnp.int32)
    ti = target_node.astype(jnp.int32)
    pos = _sc_inner(nodes_embed, context_nodes_embed, si, ti, label)
    loss = _loss(pos)
    return loss[0, 0]
